# Initial kernel scaffold; baseline (speedup 1.0000x reference)
import jax
import jax.numpy as jnp
from jax.experimental import pallas as pl


def kernel(user_word, edge_index0, edge_index1, labels, word_table, W1, b1, W2, b2):
    raise NotImplementedError("write your pallas kernel here")


# speedup vs baseline: 2.8855x; 2.8855x over previous
import functools

import jax
import jax.numpy as jnp
from jax import lax
from jax.experimental import pallas as pl
from jax.experimental.pallas import tpu as pltpu
from jax.experimental.pallas import tpu_sc as plsc

N0, N1, N2 = 10000, 5000, 1000
E0, E1 = 320000, 160000
V, D = 100000, 128
WORDS = 50

NC, NS = 2, 16
NW = NC * NS
UPT = 320
C = 128

R0S, R0D, R1S, R1D = 80, 40, 40, 8

_mesh = plsc.VectorSubcoreMesh(core_axis_name="c", subcore_axis_name="s")


def _striped(s, first_tile, tiles, n_rows, copy_fn):
    per = (-(-n_rows // tiles) + 7) // 8 * 8
    last = n_rows - (tiles - 1) * per

    def emit(base, count):
        nfull = count // 128
        rem = count - nfull * 128
        for k in range(nfull):
            copy_fn(base + k * 128, 128)
        if rem:
            copy_fn(base + nfull * 128, rem)

    if per == last:
        @pl.when(jnp.logical_and(s >= first_tile, s < first_tile + tiles))
        def _():
            emit((s - first_tile) * per, per)
    else:
        @pl.when(jnp.logical_and(s >= first_tile, s < first_tile + tiles - 1))
        def _():
            emit((s - first_tile) * per, per)

        @pl.when(s == first_tile + tiles - 1)
        def _():
            emit((tiles - 1) * per, last)


def _make_k0():
    scratch = [
        pltpu.VMEM_SHARED((R0S, D), jnp.float32),
        pltpu.VMEM_SHARED((R0D, D), jnp.float32),
        pltpu.VMEM_SHARED((R1S, D), jnp.float32),
        pltpu.VMEM_SHARED((R1D, D), jnp.float32),
        pltpu.VMEM((R0S, D), jnp.float32),
        pltpu.VMEM((R0D, D), jnp.float32),
        pltpu.VMEM((R1S, D), jnp.float32),
        pltpu.VMEM((R1D, D), jnp.float32),
        pltpu.VMEM((C,), jnp.int32),
        pltpu.VMEM((16,), jnp.int32),
        pltpu.VMEM((R0S,), jnp.int32),
        pltpu.VMEM((R0D,), jnp.int32),
        pltpu.VMEM((R1D,), jnp.int32),
    ]
    out_type = (
        jax.ShapeDtypeStruct((2 * R0S, D), jnp.float32),
        jax.ShapeDtypeStruct((2 * R0D, D), jnp.float32),
        jax.ShapeDtypeStruct((2 * R1S, D), jnp.float32),
        jax.ShapeDtypeStruct((2 * R1D, D), jnp.float32),
    )

    @functools.partial(
        pl.kernel, out_type=out_type, mesh=_mesh, scratch_types=scratch,
        name="sc_degrees",
        compiler_params=pltpu.CompilerParams(needs_layout_passes=False))
    def k0(s0_hbm, d0_hbm, s1_hbm, d1_hbm, z_hbm, r80_hbm, r40_hbm, r8_hbm,
           o0s, o0d, o1s, o1d,
           sh0s, sh0d, sh1s, sh1d, lh0s, lh0d, lh1s, lh1d,
           idx_v, idx16_v, r80_v, r40_v, r8_v):
        c = lax.axis_index("c")
        s = lax.axis_index("s")
        wid = c * NS + s

        pltpu.sync_copy(z_hbm.at[pl.ds(0, R0S)], lh0s)
        pltpu.sync_copy(z_hbm.at[pl.ds(0, R0D)], lh0d)
        pltpu.sync_copy(z_hbm.at[pl.ds(0, R1S)], lh1s)
        pltpu.sync_copy(z_hbm.at[pl.ds(0, R1D)], lh1d)
        pltpu.sync_copy(r80_hbm, r80_v)
        pltpu.sync_copy(r40_hbm, r40_v)
        pltpu.sync_copy(r8_hbm, r8_v)

        @pl.when(s == 0)
        def _():
            pltpu.sync_copy(lh0s, sh0s)
            pltpu.sync_copy(lh0d, sh0d)
            pltpu.sync_copy(lh1s, sh1s)
            pltpu.sync_copy(lh1d, sh1d)

        plsc.subcore_barrier()

        ones = jnp.ones((16,), jnp.float32)

        def scat(lh, iv, mask=None):
            plsc.addupdate_scatter(lh, [jnp.right_shift(iv, 7),
                                        jnp.bitwise_and(iv, 127)], ones,
                                   mask=mask)

        def hist(e_hbm, lh, n_per_tile):
            nfull = n_per_tile // C
            base = wid * n_per_tile

            def body(i, carry):
                pltpu.sync_copy(e_hbm.at[pl.ds(base + i * C, C)], idx_v)
                for j in range(8):
                    scat(lh, idx_v[pl.ds(j * 16, 16)])
                return carry

            lax.fori_loop(0, nfull, body, 0, unroll=False)
            tail = n_per_tile - nfull * C
            if tail:
                toff = base + nfull * C
                pltpu.sync_copy(e_hbm.at[pl.ds(toff, tail)],
                                idx16_v.at[pl.ds(0, tail)])
                iv = idx16_v[...]
                if tail == 16:
                    scat(lh, iv)
                else:
                    scat(lh, iv, mask=lax.iota(jnp.int32, 16) < tail)

        hist(s0_hbm, lh0s, E0 // NW)
        hist(d0_hbm, lh0d, E0 // NW)
        hist(s1_hbm, lh1s, E1 // NW)
        hist(d1_hbm, lh1d, E1 // NW)

        pltpu.sync_copy(lh0s, sh0s.at[r80_v], add=True)
        pltpu.sync_copy(lh0d, sh0d.at[r40_v], add=True)
        pltpu.sync_copy(lh1s, sh1s.at[r40_v], add=True)
        pltpu.sync_copy(lh1d, sh1d.at[r8_v], add=True)

        plsc.subcore_barrier()

        @pl.when(s == 0)
        def _():
            pltpu.sync_copy(sh0s, o0s.at[pl.ds(c * R0S, R0S)])

        @pl.when(s == 1)
        def _():
            pltpu.sync_copy(sh0d, o0d.at[pl.ds(c * R0D, R0D)])

        @pl.when(s == 2)
        def _():
            pltpu.sync_copy(sh1s, o1s.at[pl.ds(c * R1S, R1S)])

        @pl.when(s == 3)
        def _():
            pltpu.sync_copy(sh1d, o1d.at[pl.ds(c * R1D, R1D)])

    return k0


def _make_k1():
    scratch = [
        pltpu.VMEM_SHARED((NS * UPT, D), jnp.float32),
        pltpu.VMEM((128, D), jnp.float32),
        pltpu.VMEM((C,), jnp.int32),
        pltpu.VMEM((C,), jnp.int32),
        pltpu.VMEM((C, D), jnp.float32),
        pltpu.SemaphoreType.DMA,
    ]
    out_type = jax.ShapeDtypeStruct((N0, D), jnp.float32)

    @functools.partial(pl.kernel, out_type=out_type, mesh=_mesh,
                       scratch_types=scratch, name="sc_embed")
    def k1(wt_hbm, uw_hbm, uslot_hbm, z128_hbm, x_out,
           xacc, z128_v, idxw_v, idxu_v, rows_v, sem):
        c = lax.axis_index("c")
        s = lax.axis_index("s")
        wid = c * NS + s
        sbase = s * UPT

        pltpu.sync_copy(z128_hbm, z128_v)
        for k in range(UPT // 128):
            pltpu.sync_copy(z128_v, xacc.at[pl.ds(sbase + k * 128, 128)])
        rem = UPT - (UPT // 128) * 128
        if rem:
            pltpu.sync_copy(z128_v.at[pl.ds(0, rem)],
                            xacc.at[pl.ds(sbase + UPT - rem, rem)])

        ebase = wid * (UPT * WORDS)

        def ebody(i, carry):
            pltpu.sync_copy(uw_hbm.at[pl.ds(ebase + i * C, C)], idxw_v)
            pltpu.sync_copy(uslot_hbm.at[pl.ds(s * (UPT * WORDS) + i * C, C)],
                            idxu_v)
            pltpu.async_copy(wt_hbm.at[idxw_v], rows_v, sem).wait()
            pltpu.sync_copy(rows_v, xacc.at[idxu_v], add=True)
            return carry

        lax.fori_loop(0, (UPT * WORDS) // C, ebody, 0, unroll=False)

        @pl.when(wid < NW - 1)
        def _():
            for k in range(UPT // 128):
                pltpu.sync_copy(xacc.at[pl.ds(sbase + k * 128, 128)],
                                x_out.at[pl.ds(wid * UPT + k * 128, 128)])
            if rem:
                pltpu.sync_copy(xacc.at[pl.ds(sbase + UPT - rem, rem)],
                                x_out.at[pl.ds(wid * UPT + UPT - rem, rem)])

        @pl.when(wid == NW - 1)
        def _():
            pltpu.sync_copy(xacc.at[pl.ds(sbase, 80)],
                            x_out.at[pl.ds((NW - 1) * UPT, 80)])

    return k1


def _make_msg(n_dst, n_edges):
    per_tile = n_edges // NW
    nfull = per_tile // C
    tail = per_tile - nfull * C
    scratch = [
        pltpu.VMEM_SHARED((n_dst, D), jnp.float32),
        pltpu.VMEM((128, D), jnp.float32),
        pltpu.VMEM((C,), jnp.int32),
        pltpu.VMEM((C,), jnp.int32),
        pltpu.VMEM((16,), jnp.int32),
        pltpu.VMEM((16,), jnp.int32),
        pltpu.VMEM((8,), jnp.int32),
        pltpu.VMEM((8,), jnp.int32),
        pltpu.VMEM((C, D), jnp.float32),
        pltpu.SemaphoreType.DMA,
    ]
    out_type = jax.ShapeDtypeStruct((2 * n_dst, D), jnp.float32)

    @functools.partial(pl.kernel, out_type=out_type, mesh=_mesh,
                       scratch_types=scratch, name=f"sc_msg_{n_dst}")
    def msg(feat_hbm, src_hbm, dst_hbm, z128_hbm, out_hbm,
            agg, z128_v, sidx_v, didx_v, sidx16_v, didx16_v,
            sidx8_v, didx8_v, rows_v, sem):
        c = lax.axis_index("c")
        s = lax.axis_index("s")
        wid = c * NS + s

        pltpu.sync_copy(z128_hbm, z128_v)
        _striped(s, 0, 8, n_dst,
                 lambda off, sz: pltpu.sync_copy(z128_v.at[pl.ds(0, sz)],
                                                 agg.at[pl.ds(off, sz)]))
        plsc.subcore_barrier()

        base = wid * per_tile

        def body(i, carry):
            off = base + i * C
            pltpu.sync_copy(src_hbm.at[pl.ds(off, C)], sidx_v)
            pltpu.sync_copy(dst_hbm.at[pl.ds(off, C)], didx_v)
            pltpu.async_copy(feat_hbm.at[sidx_v], rows_v, sem).wait()
            pltpu.sync_copy(rows_v, agg.at[didx_v], add=True)
            return carry

        lax.fori_loop(0, nfull, body, 0, unroll=False)

        toff = base + nfull * C
        if tail == 16:
            pltpu.sync_copy(src_hbm.at[pl.ds(toff, 16)], sidx16_v)
            pltpu.sync_copy(dst_hbm.at[pl.ds(toff, 16)], didx16_v)
            pltpu.async_copy(feat_hbm.at[sidx16_v], rows_v.at[pl.ds(0, 16)], sem).wait()
            pltpu.sync_copy(rows_v.at[pl.ds(0, 16)], agg.at[didx16_v], add=True)
        elif tail == 8:
            pltpu.sync_copy(src_hbm.at[pl.ds(toff, 8)], sidx8_v)
            pltpu.sync_copy(dst_hbm.at[pl.ds(toff, 8)], didx8_v)
            pltpu.async_copy(feat_hbm.at[sidx8_v], rows_v.at[pl.ds(0, 8)], sem).wait()
            pltpu.sync_copy(rows_v.at[pl.ds(0, 8)], agg.at[didx8_v], add=True)

        plsc.subcore_barrier()

        _striped(s, 0, 8, n_dst,
                 lambda off, sz: pltpu.sync_copy(
                     agg.at[pl.ds(off, sz)],
                     out_hbm.at[pl.ds(c * n_dst + off, sz)]))

    return msg


_k0 = _make_k0()
_k1 = _make_k1()
_msg0 = _make_msg(N1, E0)
_msg1 = _make_msg(N2, E1)


def _scale_body(x_ref, d_ref, o_ref):
    d = jnp.maximum(d_ref[...], 1.0)
    o_ref[...] = x_ref[...] * (lax.rsqrt(d) * (1.0 / WORDS))


def _tc_scale(x, deg):
    grid = (N0 // 1000,)
    return pl.pallas_call(
        _scale_body,
        grid=grid,
        in_specs=[pl.BlockSpec((1000, D), lambda i: (i, 0)),
                  pl.BlockSpec((1000, 1), lambda i: (i, 0))],
        out_specs=pl.BlockSpec((1000, D), lambda i: (i, 0)),
        out_shape=jax.ShapeDtypeStruct((N0, D), jnp.float32),
    )(x, deg)


def _layer_body(scale_next, p0_ref, p1_ref, dd_ref, ds_ref, w_ref, b_ref, o_ref):
    dd = jnp.maximum(dd_ref[...], 1.0)
    agg = (p0_ref[...] + p1_ref[...]) * lax.rsqrt(dd)
    h = jnp.dot(agg, w_ref[...], preferred_element_type=jnp.float32) + b_ref[...]
    h = jnp.maximum(h, 0.0)
    if scale_next:
        ds = jnp.maximum(ds_ref[...], 1.0)
        h = h * lax.rsqrt(ds)
    o_ref[...] = h


def _tc_layer(p0, p1, deg_dst, deg_src_next, w, b, scale_next):
    n = p0.shape[0]
    blk = 1000
    grid = (n // blk,)
    return pl.pallas_call(
        functools.partial(_layer_body, scale_next),
        grid=grid,
        in_specs=[pl.BlockSpec((blk, D), lambda i: (i, 0)),
                  pl.BlockSpec((blk, D), lambda i: (i, 0)),
                  pl.BlockSpec((blk, 1), lambda i: (i, 0)),
                  pl.BlockSpec((blk, 1), lambda i: (i, 0)),
                  pl.BlockSpec((D, D), lambda i: (0, 0)),
                  pl.BlockSpec((1, D), lambda i: (0, 0))],
        out_specs=pl.BlockSpec((blk, D), lambda i: (i, 0)),
        out_shape=jax.ShapeDtypeStruct((n, D), jnp.float32),
    )(p0, p1, deg_dst, deg_src_next, w, b)


def _hist_to_deg(hp, rows, n):
    return (hp[:rows] + hp[rows:]).reshape(-1)[:n].reshape(n, 1)


def kernel(user_word, edge_index0, edge_index1, labels, word_table, W1, b1, W2, b2):
    uw_flat = user_word.reshape(-1).astype(jnp.int32)
    uw_pad = jnp.concatenate(
        [uw_flat, jnp.zeros((NW * UPT * WORDS - uw_flat.shape[0],), jnp.int32)])
    uslot = ((jnp.arange(UPT * WORDS, dtype=jnp.int32) // WORDS)[None, :]
             + jnp.arange(NS, dtype=jnp.int32)[:, None] * UPT).reshape(-1)

    src0 = edge_index0[0].astype(jnp.int32)
    dst0 = edge_index0[1].astype(jnp.int32)
    src1 = edge_index1[0].astype(jnp.int32)
    dst1 = edge_index1[1].astype(jnp.int32)

    z128 = jnp.zeros((128, D), jnp.float32)
    r80 = jnp.arange(R0S, dtype=jnp.int32)
    r40 = jnp.arange(R0D, dtype=jnp.int32)
    r8 = jnp.arange(R1D, dtype=jnp.int32)

    hp0s, hp0d, hp1s, hp1d = _k0(src0, dst0, src1, dst1, z128, r80, r40, r8)
    x_sum = _k1(word_table, uw_pad, uslot, z128)

    deg0s = _hist_to_deg(hp0s, R0S, N0)
    deg0d = _hist_to_deg(hp0d, R0D, N1)
    deg1s = _hist_to_deg(hp1s, R1S, N1)
    deg1d = _hist_to_deg(hp1d, R1D, N2)

    feat0 = _tc_scale(x_sum, deg0s)

    agg0 = _msg0(feat0, src0, dst0, z128)
    feat1 = _tc_layer(agg0[:N1], agg0[N1:], deg0d, deg1s, W1,
                      b1.reshape(1, D), True)

    agg1 = _msg1(feat1, src1, dst1, z128)
    out = _tc_layer(agg1[:N2], agg1[N2:], deg1d, deg1d, W2,
                    b2.reshape(1, D), False)

    return (out, labels)


# speedup vs baseline: 3.6671x; 1.2709x over previous
import functools

import jax
import jax.numpy as jnp
from jax import lax
from jax.experimental import pallas as pl
from jax.experimental.pallas import tpu as pltpu
from jax.experimental.pallas import tpu_sc as plsc

N0, N1, N2 = 10000, 5000, 1000
E0, E1 = 320000, 160000
V, D = 100000, 128
WORDS = 50

NC, NS = 2, 16
NW = NC * NS
UPT = 320
C = 128

R0S, R0D, R1S, R1D = 80, 40, 40, 8

_mesh = plsc.VectorSubcoreMesh(core_axis_name="c", subcore_axis_name="s")


def _striped(s, first_tile, tiles, n_rows, copy_fn):
    per = (-(-n_rows // tiles) + 7) // 8 * 8
    last = n_rows - (tiles - 1) * per

    def emit(base, count):
        nfull = count // 128
        rem = count - nfull * 128
        for k in range(nfull):
            copy_fn(base + k * 128, 128)
        if rem:
            copy_fn(base + nfull * 128, rem)

    if per == last:
        @pl.when(jnp.logical_and(s >= first_tile, s < first_tile + tiles))
        def _():
            emit((s - first_tile) * per, per)
    else:
        @pl.when(jnp.logical_and(s >= first_tile, s < first_tile + tiles - 1))
        def _():
            emit((s - first_tile) * per, per)

        @pl.when(s == first_tile + tiles - 1)
        def _():
            emit((tiles - 1) * per, last)


def _make_k0():
    scratch = [
        pltpu.VMEM_SHARED((R0S, D), jnp.float32),
        pltpu.VMEM_SHARED((R0D, D), jnp.float32),
        pltpu.VMEM_SHARED((R1S, D), jnp.float32),
        pltpu.VMEM_SHARED((R1D, D), jnp.float32),
        pltpu.VMEM((R0S, D), jnp.float32),
        pltpu.VMEM((R0D, D), jnp.float32),
        pltpu.VMEM((R1S, D), jnp.float32),
        pltpu.VMEM((R1D, D), jnp.float32),
        pltpu.VMEM((C,), jnp.int32),
        pltpu.VMEM((16,), jnp.int32),
        pltpu.VMEM((R0S,), jnp.int32),
        pltpu.VMEM((R0D,), jnp.int32),
        pltpu.VMEM((R1D,), jnp.int32),
    ]
    out_type = (
        jax.ShapeDtypeStruct((2 * R0S, D), jnp.float32),
        jax.ShapeDtypeStruct((2 * R0D, D), jnp.float32),
        jax.ShapeDtypeStruct((2 * R1S, D), jnp.float32),
        jax.ShapeDtypeStruct((2 * R1D, D), jnp.float32),
    )

    @functools.partial(
        pl.kernel, out_type=out_type, mesh=_mesh, scratch_types=scratch,
        name="sc_degrees",
        compiler_params=pltpu.CompilerParams(needs_layout_passes=False))
    def k0(s0_hbm, d0_hbm, s1_hbm, d1_hbm, z_hbm, r80_hbm, r40_hbm, r8_hbm,
           o0s, o0d, o1s, o1d,
           sh0s, sh0d, sh1s, sh1d, lh0s, lh0d, lh1s, lh1d,
           idx_v, idx16_v, r80_v, r40_v, r8_v):
        c = lax.axis_index("c")
        s = lax.axis_index("s")
        wid = c * NS + s

        pltpu.sync_copy(z_hbm.at[pl.ds(0, R0S)], lh0s)
        pltpu.sync_copy(z_hbm.at[pl.ds(0, R0D)], lh0d)
        pltpu.sync_copy(z_hbm.at[pl.ds(0, R1S)], lh1s)
        pltpu.sync_copy(z_hbm.at[pl.ds(0, R1D)], lh1d)
        pltpu.sync_copy(r80_hbm, r80_v)
        pltpu.sync_copy(r40_hbm, r40_v)
        pltpu.sync_copy(r8_hbm, r8_v)

        @pl.when(s == 0)
        def _():
            pltpu.sync_copy(lh0s, sh0s)
            pltpu.sync_copy(lh0d, sh0d)
            pltpu.sync_copy(lh1s, sh1s)
            pltpu.sync_copy(lh1d, sh1d)

        plsc.subcore_barrier()

        ones = jnp.ones((16,), jnp.float32)

        def scat(lh, iv, mask=None):
            plsc.addupdate_scatter(lh, [jnp.right_shift(iv, 7),
                                        jnp.bitwise_and(iv, 127)], ones,
                                   mask=mask)

        def hist(e_hbm, lh, n_per_tile):
            nfull = n_per_tile // C
            base = wid * n_per_tile

            def body(i, carry):
                pltpu.sync_copy(e_hbm.at[pl.ds(base + i * C, C)], idx_v)
                for j in range(8):
                    scat(lh, idx_v[pl.ds(j * 16, 16)])
                return carry

            lax.fori_loop(0, nfull, body, 0, unroll=False)
            tail = n_per_tile - nfull * C
            if tail:
                toff = base + nfull * C
                pltpu.sync_copy(e_hbm.at[pl.ds(toff, tail)],
                                idx16_v.at[pl.ds(0, tail)])
                iv = idx16_v[...]
                if tail == 16:
                    scat(lh, iv)
                else:
                    scat(lh, iv, mask=lax.iota(jnp.int32, 16) < tail)

        hist(s0_hbm, lh0s, E0 // NW)
        hist(d0_hbm, lh0d, E0 // NW)
        hist(s1_hbm, lh1s, E1 // NW)
        hist(d1_hbm, lh1d, E1 // NW)

        pltpu.sync_copy(lh0s, sh0s.at[r80_v], add=True)
        pltpu.sync_copy(lh0d, sh0d.at[r40_v], add=True)
        pltpu.sync_copy(lh1s, sh1s.at[r40_v], add=True)
        pltpu.sync_copy(lh1d, sh1d.at[r8_v], add=True)

        plsc.subcore_barrier()

        @pl.when(s == 0)
        def _():
            pltpu.sync_copy(sh0s, o0s.at[pl.ds(c * R0S, R0S)])

        @pl.when(s == 1)
        def _():
            pltpu.sync_copy(sh0d, o0d.at[pl.ds(c * R0D, R0D)])

        @pl.when(s == 2)
        def _():
            pltpu.sync_copy(sh1s, o1s.at[pl.ds(c * R1S, R1S)])

        @pl.when(s == 3)
        def _():
            pltpu.sync_copy(sh1d, o1d.at[pl.ds(c * R1D, R1D)])

    return k0


def _make_k1():
    scratch = [
        pltpu.VMEM_SHARED((NS * UPT, D), jnp.float32),
        pltpu.VMEM((128, D), jnp.float32),
        pltpu.VMEM((C,), jnp.int32),
        pltpu.VMEM((C,), jnp.int32),
        pltpu.VMEM((C,), jnp.int32),
        pltpu.VMEM((C,), jnp.int32),
        pltpu.VMEM((C, D), jnp.float32),
        pltpu.VMEM((C, D), jnp.float32),
        pltpu.SemaphoreType.DMA,
        pltpu.SemaphoreType.DMA,
    ]
    out_type = jax.ShapeDtypeStruct((N0, D), jnp.float32)

    @functools.partial(pl.kernel, out_type=out_type, mesh=_mesh,
                       scratch_types=scratch, name="sc_embed")
    def k1(wt_hbm, uw_hbm, uslot_hbm, z128_hbm, x_out,
           xacc, z128_v, idxw0, idxw1, idxu0, idxu1, rows0, rows1,
           sem0, sem1):
        c = lax.axis_index("c")
        s = lax.axis_index("s")
        wid = c * NS + s
        sbase = s * UPT

        pltpu.sync_copy(z128_hbm, z128_v)
        for k in range(UPT // 128):
            pltpu.sync_copy(z128_v, xacc.at[pl.ds(sbase + k * 128, 128)])
        rem = UPT - (UPT // 128) * 128
        if rem:
            pltpu.sync_copy(z128_v.at[pl.ds(0, rem)],
                            xacc.at[pl.ds(sbase + UPT - rem, rem)])

        ebase = wid * (UPT * WORDS)
        ubase = s * (UPT * WORDS)
        n = (UPT * WORDS) // C

        def issue(i, idxw, idxu, rows, sem):
            pltpu.sync_copy(uw_hbm.at[pl.ds(ebase + i * C, C)], idxw)
            pltpu.sync_copy(uslot_hbm.at[pl.ds(ubase + i * C, C)], idxu)
            pltpu.async_copy(wt_hbm.at[idxw], rows, sem)

        def wait_g(idxw, rows, sem):
            pltpu.make_async_copy(wt_hbm.at[idxw], rows, sem).wait()

        issue(0, idxw0, idxu0, rows0, sem0)

        def gbody(g, carry):
            c0 = 2 * g
            issue(c0 + 1, idxw1, idxu1, rows1, sem1)
            wait_g(idxw0, rows0, sem0)
            pltpu.sync_copy(rows0, xacc.at[idxu0], add=True)
            issue(c0 + 2, idxw0, idxu0, rows0, sem0)
            wait_g(idxw1, rows1, sem1)
            pltpu.sync_copy(rows1, xacc.at[idxu1], add=True)
            return carry

        lax.fori_loop(0, (n - 1) // 2, gbody, 0, unroll=False)
        wait_g(idxw0, rows0, sem0)
        pltpu.sync_copy(rows0, xacc.at[idxu0], add=True)

        @pl.when(wid < NW - 1)
        def _():
            for k in range(UPT // 128):
                pltpu.sync_copy(xacc.at[pl.ds(sbase + k * 128, 128)],
                                x_out.at[pl.ds(wid * UPT + k * 128, 128)])
            if rem:
                pltpu.sync_copy(xacc.at[pl.ds(sbase + UPT - rem, rem)],
                                x_out.at[pl.ds(wid * UPT + UPT - rem, rem)])

        @pl.when(wid == NW - 1)
        def _():
            pltpu.sync_copy(xacc.at[pl.ds(sbase, 80)],
                            x_out.at[pl.ds((NW - 1) * UPT, 80)])

    return k1


def _make_msg(n_dst, n_edges):
    per_tile = n_edges // NW
    nfull = per_tile // C
    tail = per_tile - nfull * C
    scratch = [
        pltpu.VMEM_SHARED((n_dst, D), jnp.float32),
        pltpu.VMEM((128, D), jnp.float32),
        pltpu.VMEM((C,), jnp.int32),
        pltpu.VMEM((C,), jnp.int32),
        pltpu.VMEM((C,), jnp.int32),
        pltpu.VMEM((C,), jnp.int32),
        pltpu.VMEM((16,), jnp.int32),
        pltpu.VMEM((16,), jnp.int32),
        pltpu.VMEM((8,), jnp.int32),
        pltpu.VMEM((8,), jnp.int32),
        pltpu.VMEM((C, D), jnp.float32),
        pltpu.VMEM((C, D), jnp.float32),
        pltpu.SemaphoreType.DMA,
        pltpu.SemaphoreType.DMA,
    ]
    out_type = jax.ShapeDtypeStruct((2 * n_dst, D), jnp.float32)

    @functools.partial(pl.kernel, out_type=out_type, mesh=_mesh,
                       scratch_types=scratch, name=f"sc_msg_{n_dst}")
    def msg(feat_hbm, src_hbm, dst_hbm, z128_hbm, out_hbm,
            agg, z128_v, sidx0, sidx1, didx0, didx1, sidx16_v, didx16_v,
            sidx8_v, didx8_v, rows0, rows1, sem0, sem1):
        c = lax.axis_index("c")
        s = lax.axis_index("s")
        wid = c * NS + s

        pltpu.sync_copy(z128_hbm, z128_v)
        _striped(s, 0, 8, n_dst,
                 lambda off, sz: pltpu.sync_copy(z128_v.at[pl.ds(0, sz)],
                                                 agg.at[pl.ds(off, sz)]))
        plsc.subcore_barrier()

        base = wid * per_tile

        def issue(i, sidx, didx, rows, sem):
            pltpu.sync_copy(src_hbm.at[pl.ds(base + i * C, C)], sidx)
            pltpu.sync_copy(dst_hbm.at[pl.ds(base + i * C, C)], didx)
            pltpu.async_copy(feat_hbm.at[sidx], rows, sem)

        def wait_g(sidx, rows, sem):
            pltpu.make_async_copy(feat_hbm.at[sidx], rows, sem).wait()

        issue(0, sidx0, didx0, rows0, sem0)

        def gbody(g, carry):
            c0 = 2 * g
            issue(c0 + 1, sidx1, didx1, rows1, sem1)
            wait_g(sidx0, rows0, sem0)
            pltpu.sync_copy(rows0, agg.at[didx0], add=True)
            issue(jnp.minimum(c0 + 2, nfull - 1), sidx0, didx0, rows0, sem0)
            wait_g(sidx1, rows1, sem1)
            pltpu.sync_copy(rows1, agg.at[didx1], add=True)
            return carry

        lax.fori_loop(0, nfull // 2, gbody, 0, unroll=False)
        wait_g(sidx0, rows0, sem0)
        if nfull % 2:
            pltpu.sync_copy(rows0, agg.at[didx0], add=True)

        toff = base + nfull * C
        if tail == 16:
            pltpu.sync_copy(src_hbm.at[pl.ds(toff, 16)], sidx16_v)
            pltpu.sync_copy(dst_hbm.at[pl.ds(toff, 16)], didx16_v)
            pltpu.async_copy(feat_hbm.at[sidx16_v], rows0.at[pl.ds(0, 16)], sem0).wait()
            pltpu.sync_copy(rows0.at[pl.ds(0, 16)], agg.at[didx16_v], add=True)
        elif tail == 8:
            pltpu.sync_copy(src_hbm.at[pl.ds(toff, 8)], sidx8_v)
            pltpu.sync_copy(dst_hbm.at[pl.ds(toff, 8)], didx8_v)
            pltpu.async_copy(feat_hbm.at[sidx8_v], rows0.at[pl.ds(0, 8)], sem0).wait()
            pltpu.sync_copy(rows0.at[pl.ds(0, 8)], agg.at[didx8_v], add=True)

        plsc.subcore_barrier()

        _striped(s, 0, 8, n_dst,
                 lambda off, sz: pltpu.sync_copy(
                     agg.at[pl.ds(off, sz)],
                     out_hbm.at[pl.ds(c * n_dst + off, sz)]))

    return msg


_k0 = _make_k0()
_k1 = _make_k1()
_msg0 = _make_msg(N1, E0)
_msg1 = _make_msg(N2, E1)


def _scale_body(x_ref, d_ref, o_ref):
    d = jnp.maximum(d_ref[...], 1.0)
    o_ref[...] = x_ref[...] * (lax.rsqrt(d) * (1.0 / WORDS))


def _tc_scale(x, deg):
    grid = (N0 // 1000,)
    return pl.pallas_call(
        _scale_body,
        grid=grid,
        in_specs=[pl.BlockSpec((1000, D), lambda i: (i, 0)),
                  pl.BlockSpec((1000, 1), lambda i: (i, 0))],
        out_specs=pl.BlockSpec((1000, D), lambda i: (i, 0)),
        out_shape=jax.ShapeDtypeStruct((N0, D), jnp.float32),
    )(x, deg)


def _layer_body(scale_next, p0_ref, p1_ref, dd_ref, ds_ref, w_ref, b_ref, o_ref):
    dd = jnp.maximum(dd_ref[...], 1.0)
    agg = (p0_ref[...] + p1_ref[...]) * lax.rsqrt(dd)
    h = jnp.dot(agg, w_ref[...], preferred_element_type=jnp.float32) + b_ref[...]
    h = jnp.maximum(h, 0.0)
    if scale_next:
        ds = jnp.maximum(ds_ref[...], 1.0)
        h = h * lax.rsqrt(ds)
    o_ref[...] = h


def _tc_layer(p0, p1, deg_dst, deg_src_next, w, b, scale_next):
    n = p0.shape[0]
    blk = 1000
    grid = (n // blk,)
    return pl.pallas_call(
        functools.partial(_layer_body, scale_next),
        grid=grid,
        in_specs=[pl.BlockSpec((blk, D), lambda i: (i, 0)),
                  pl.BlockSpec((blk, D), lambda i: (i, 0)),
                  pl.BlockSpec((blk, 1), lambda i: (i, 0)),
                  pl.BlockSpec((blk, 1), lambda i: (i, 0)),
                  pl.BlockSpec((D, D), lambda i: (0, 0)),
                  pl.BlockSpec((1, D), lambda i: (0, 0))],
        out_specs=pl.BlockSpec((blk, D), lambda i: (i, 0)),
        out_shape=jax.ShapeDtypeStruct((n, D), jnp.float32),
    )(p0, p1, deg_dst, deg_src_next, w, b)


def _hist_to_deg(hp, rows, n):
    return (hp[:rows] + hp[rows:]).reshape(-1)[:n].reshape(n, 1)


def kernel(user_word, edge_index0, edge_index1, labels, word_table, W1, b1, W2, b2):
    uw_flat = user_word.reshape(-1).astype(jnp.int32)
    uw_pad = jnp.concatenate(
        [uw_flat, jnp.zeros((NW * UPT * WORDS - uw_flat.shape[0],), jnp.int32)])
    uslot = ((jnp.arange(UPT * WORDS, dtype=jnp.int32) // WORDS)[None, :]
             + jnp.arange(NS, dtype=jnp.int32)[:, None] * UPT).reshape(-1)

    src0 = edge_index0[0].astype(jnp.int32)
    dst0 = edge_index0[1].astype(jnp.int32)
    src1 = edge_index1[0].astype(jnp.int32)
    dst1 = edge_index1[1].astype(jnp.int32)

    z128 = jnp.zeros((128, D), jnp.float32)
    r80 = jnp.arange(R0S, dtype=jnp.int32)
    r40 = jnp.arange(R0D, dtype=jnp.int32)
    r8 = jnp.arange(R1D, dtype=jnp.int32)

    hp0s, hp0d, hp1s, hp1d = _k0(src0, dst0, src1, dst1, z128, r80, r40, r8)
    x_sum = _k1(word_table, uw_pad, uslot, z128)

    deg0s = _hist_to_deg(hp0s, R0S, N0)
    deg0d = _hist_to_deg(hp0d, R0D, N1)
    deg1s = _hist_to_deg(hp1s, R1S, N1)
    deg1d = _hist_to_deg(hp1d, R1D, N2)

    feat0 = _tc_scale(x_sum, deg0s)

    agg0 = _msg0(feat0, src0, dst0, z128)
    feat1 = _tc_layer(agg0[:N1], agg0[N1:], deg0d, deg1s, W1,
                      b1.reshape(1, D), True)

    agg1 = _msg1(feat1, src1, dst1, z128)
    out = _tc_layer(agg1[:N2], agg1[N2:], deg1d, deg1d, W2,
                    b2.reshape(1, D), False)

    return (out, labels)


# speedup vs baseline: 7.5684x; 2.0639x over previous
import functools

import jax
import jax.numpy as jnp
import numpy as np
from jax import lax
from jax.experimental import pallas as pl
from jax.experimental.pallas import tpu as pltpu
from jax.experimental.pallas import tpu_sc as plsc

N0, N1, N2 = 10000, 5000, 1000
E0, E1 = 320000, 160000
V, D = 100000, 128
WORDS = 50

NC, NS = 2, 16
NW = NC * NS
UPT = 320
C = 128

R0S, R0D, R1S, R1D = 80, 40, 40, 8

_mesh = plsc.VectorSubcoreMesh(core_axis_name="c", subcore_axis_name="s")


def _striped(s, first_tile, tiles, n_rows, copy_fn, chunk=128):
    per = (-(-n_rows // tiles) + 7) // 8 * 8
    last = n_rows - (tiles - 1) * per

    def emit(base, count):
        nfull = count // chunk
        rem = count - nfull * chunk
        for k in range(nfull):
            copy_fn(base + k * chunk, chunk)
        if rem:
            copy_fn(base + nfull * chunk, rem)

    if per == last:
        @pl.when(jnp.logical_and(s >= first_tile, s < first_tile + tiles))
        def _():
            emit((s - first_tile) * per, per)
    else:
        @pl.when(jnp.logical_and(s >= first_tile, s < first_tile + tiles - 1))
        def _():
            emit((s - first_tile) * per, per)

        @pl.when(s == first_tile + tiles - 1)
        def _():
            emit((tiles - 1) * per, last)


def _ring(n, nb, d, issue, wait_gather, scat_async, wait_scat):
    for j in range(d):
        issue(j, j)
    G = n // nb

    def gbody(g, carry):
        for b in range(nb):
            i = nb * g + b
            wait_gather(b)
            scat_async(b)
            jj = i + d
            bj = (b + d) % nb

            @pl.when(jj < n)
            def _():
                wait_scat(bj)
                issue(jj, bj)
        return carry

    lax.fori_loop(0, G, gbody, 0, unroll=False)
    for k in range(n - G * nb):
        i = G * nb + k
        b = i % nb
        wait_gather(b)
        scat_async(b)
        jj = i + d
        if jj < n:
            bj = jj % nb
            wait_scat(bj)
            issue(jj, bj)
    for k in range(min(nb, n)):
        wait_scat((n - 1 - k) % nb)


def _make_k0():
    scratch = [
        pltpu.VMEM_SHARED((R0S, D), jnp.float32),
        pltpu.VMEM_SHARED((R0D, D), jnp.float32),
        pltpu.VMEM_SHARED((R1S, D), jnp.float32),
        pltpu.VMEM_SHARED((R1D, D), jnp.float32),
        pltpu.VMEM((R0S, D), jnp.float32),
        pltpu.VMEM((R0D, D), jnp.float32),
        pltpu.VMEM((R1S, D), jnp.float32),
        pltpu.VMEM((R1D, D), jnp.float32),
        pltpu.VMEM((E0 // NW,), jnp.int32),
        pltpu.VMEM((E0 // NW,), jnp.int32),
        pltpu.VMEM((R0S,), jnp.int32),
        pltpu.VMEM((R0D,), jnp.int32),
        pltpu.VMEM((R1D,), jnp.int32),
        pltpu.SemaphoreType.DMA,
        pltpu.SemaphoreType.DMA,
    ]
    out_type = (
        jax.ShapeDtypeStruct((2 * R0S, D), jnp.float32),
        jax.ShapeDtypeStruct((2 * R0D, D), jnp.float32),
        jax.ShapeDtypeStruct((2 * R1S, D), jnp.float32),
        jax.ShapeDtypeStruct((2 * R1D, D), jnp.float32),
    )

    @functools.partial(
        pl.kernel, out_type=out_type, mesh=_mesh, scratch_types=scratch,
        name="sc_degrees",
        compiler_params=pltpu.CompilerParams(needs_layout_passes=False))
    def k0(s0_hbm, d0_hbm, s1_hbm, d1_hbm, z_hbm, r80_hbm, r40_hbm, r8_hbm,
           o0s, o0d, o1s, o1d,
           sh0s, sh0d, sh1s, sh1d, lh0s, lh0d, lh1s, lh1d,
           ib0, ib1, r80_v, r40_v, r8_v, semA, semB):
        c = lax.axis_index("c")
        s = lax.axis_index("s")
        wid = c * NS + s

        pltpu.sync_copy(z_hbm.at[pl.ds(0, R0S)], lh0s)
        pltpu.sync_copy(z_hbm.at[pl.ds(0, R0D)], lh0d)
        pltpu.sync_copy(z_hbm.at[pl.ds(0, R1S)], lh1s)
        pltpu.sync_copy(z_hbm.at[pl.ds(0, R1D)], lh1d)
        pltpu.sync_copy(r80_hbm, r80_v)
        pltpu.sync_copy(r40_hbm, r40_v)
        pltpu.sync_copy(r8_hbm, r8_v)

        @pl.when(s == 0)
        def _():
            pltpu.sync_copy(lh0s, sh0s)
            pltpu.sync_copy(lh0d, sh0d)
            pltpu.sync_copy(lh1s, sh1s)
            pltpu.sync_copy(lh1d, sh1d)

        plsc.subcore_barrier()

        ones = jnp.ones((16,), jnp.float32)

        def scat(lh, iv, mask=None):
            plsc.addupdate_scatter(lh, [jnp.right_shift(iv, 7),
                                        jnp.bitwise_and(iv, 127)], ones,
                                   mask=mask)

        def process(lh, ib, size):
            nvec = size // 16

            def vbody(v, carry):
                scat(lh, ib[pl.ds(v * 16, 16)])
                return carry

            lax.fori_loop(0, nvec, vbody, 0, unroll=False)
            vtail = size - nvec * 16
            if vtail:
                iv = ib[pl.ds(nvec * 16, 16)]
                scat(lh, iv, mask=lax.iota(jnp.int32, 16) < vtail)

        jobs = [(s0_hbm, lh0s, E0 // NW), (d0_hbm, lh0d, E0 // NW),
                (s1_hbm, lh1s, E1 // NW), (d1_hbm, lh1d, E1 // NW)]
        bufs = (ib0, ib1)
        semx = (semA, semB)

        def load(k):
            e_hbm, _, n = jobs[k]
            pltpu.async_copy(e_hbm.at[pl.ds(wid * n, n)],
                             bufs[k % 2].at[pl.ds(0, n)], semx[k % 2])

        def wait(k):
            e_hbm, _, n = jobs[k]
            pltpu.make_async_copy(e_hbm.at[pl.ds(wid * n, n)],
                                  bufs[k % 2].at[pl.ds(0, n)],
                                  semx[k % 2]).wait()

        load(0)
        load(1)
        for k in range(len(jobs)):
            wait(k)
            process(jobs[k][1], bufs[k % 2], jobs[k][2])
            if k + 2 < len(jobs):
                load(k + 2)

        pltpu.sync_copy(lh0s, sh0s.at[r80_v], add=True)
        pltpu.sync_copy(lh0d, sh0d.at[r40_v], add=True)
        pltpu.sync_copy(lh1s, sh1s.at[r40_v], add=True)
        pltpu.sync_copy(lh1d, sh1d.at[r8_v], add=True)

        plsc.subcore_barrier()

        @pl.when(s == 0)
        def _():
            pltpu.sync_copy(sh0s, o0s.at[pl.ds(c * R0S, R0S)])

        @pl.when(s == 1)
        def _():
            pltpu.sync_copy(sh0d, o0d.at[pl.ds(c * R0D, R0D)])

        @pl.when(s == 2)
        def _():
            pltpu.sync_copy(sh1s, o1s.at[pl.ds(c * R1S, R1S)])

        @pl.when(s == 3)
        def _():
            pltpu.sync_copy(sh1d, o1d.at[pl.ds(c * R1D, R1D)])

    return k0


def _make_k1():
    NB = 5
    ZR = 40
    scratch = [
        pltpu.VMEM_SHARED((NS * UPT, D), jnp.float32),
        pltpu.VMEM((ZR, D), jnp.float32),
        [pltpu.VMEM((C,), jnp.int32) for _ in range(NB)],
        [pltpu.VMEM((C,), jnp.int32) for _ in range(NB)],
        [pltpu.VMEM((C, D), jnp.float32) for _ in range(NB)],
        [pltpu.SemaphoreType.DMA for _ in range(NB)],
        [pltpu.SemaphoreType.DMA for _ in range(NB)],
    ]
    out_type = jax.ShapeDtypeStruct((N0, D), jnp.float32)

    @functools.partial(pl.kernel, out_type=out_type, mesh=_mesh,
                       scratch_types=scratch, name="sc_embed")
    def k1(wt_hbm, uw_hbm, uslot_hbm, z128_hbm, x_out,
           xacc, z128_v, idxw, idxu, rows, semg, sems):
        c = lax.axis_index("c")
        s = lax.axis_index("s")
        wid = c * NS + s
        sbase = s * UPT

        pltpu.sync_copy(z128_hbm.at[pl.ds(0, ZR)], z128_v)
        for k in range(UPT // ZR):
            pltpu.sync_copy(z128_v, xacc.at[pl.ds(sbase + k * ZR, ZR)])

        ebase = jnp.minimum(wid * (UPT * WORDS), N0 * WORDS - UPT * WORDS)
        ubase = wid * (UPT * WORDS)
        n = (UPT * WORDS) // C

        def issue(i, b):
            pltpu.sync_copy(uw_hbm.at[pl.ds(ebase + i * C, C)], idxw[b])
            pltpu.sync_copy(uslot_hbm.at[pl.ds(ubase + i * C, C)], idxu[b])
            pltpu.async_copy(wt_hbm.at[idxw[b]], rows[b], semg[b])

        _ring(n, NB, 3, issue,
              lambda b: pltpu.make_async_copy(wt_hbm.at[idxw[b]], rows[b],
                                              semg[b]).wait(),
              lambda b: pltpu.sync_copy(rows[b], xacc.at[idxu[b]], add=True),
              lambda b: None)

        @pl.when(wid < NW - 1)
        def _():
            for k in range(UPT // 128):
                pltpu.sync_copy(xacc.at[pl.ds(sbase + k * 128, 128)],
                                x_out.at[pl.ds(wid * UPT + k * 128, 128)])
            rem = UPT - (UPT // 128) * 128
            if rem:
                pltpu.sync_copy(xacc.at[pl.ds(sbase + UPT - rem, rem)],
                                x_out.at[pl.ds(wid * UPT + UPT - rem, rem)])

        @pl.when(wid == NW - 1)
        def _():
            pltpu.sync_copy(xacc.at[pl.ds(sbase, 80)],
                            x_out.at[pl.ds((NW - 1) * UPT, 80)])

    return k1


def _make_msg(n_dst, n_edges):
    per_tile = n_edges // NW
    nfull = per_tile // C
    tail = per_tile - nfull * C
    NB = 5
    ZR = 40
    scratch = [
        pltpu.VMEM_SHARED((n_dst, D), jnp.float32),
        pltpu.VMEM((ZR, D), jnp.float32),
        [pltpu.VMEM((C,), jnp.int32) for _ in range(NB)],
        [pltpu.VMEM((C,), jnp.int32) for _ in range(NB)],
        pltpu.VMEM((16,), jnp.int32),
        pltpu.VMEM((16,), jnp.int32),
        pltpu.VMEM((8,), jnp.int32),
        pltpu.VMEM((8,), jnp.int32),
        [pltpu.VMEM((C, D), jnp.float32) for _ in range(NB)],
        [pltpu.SemaphoreType.DMA for _ in range(NB)],
        [pltpu.SemaphoreType.DMA for _ in range(NB)],
    ]
    out_type = jax.ShapeDtypeStruct((2 * n_dst, D), jnp.float32)

    @functools.partial(pl.kernel, out_type=out_type, mesh=_mesh,
                       scratch_types=scratch, name=f"sc_msg_{n_dst}")
    def msg(feat_hbm, src_hbm, dst_hbm, z128_hbm, out_hbm,
            agg, z128_v, sidx, didx, sidx16_v, didx16_v,
            sidx8_v, didx8_v, rows, semg, sems):
        c = lax.axis_index("c")
        s = lax.axis_index("s")
        wid = c * NS + s

        pltpu.sync_copy(z128_hbm.at[pl.ds(0, ZR)], z128_v)
        _striped(s, 0, 8, n_dst,
                 lambda off, sz: pltpu.sync_copy(z128_v.at[pl.ds(0, sz)],
                                                 agg.at[pl.ds(off, sz)]),
                 chunk=ZR)
        plsc.subcore_barrier()

        base = wid * per_tile

        def issue(i, b):
            pltpu.sync_copy(src_hbm.at[pl.ds(base + i * C, C)], sidx[b])
            pltpu.sync_copy(dst_hbm.at[pl.ds(base + i * C, C)], didx[b])
            pltpu.async_copy(feat_hbm.at[sidx[b]], rows[b], semg[b])

        _ring(nfull, NB, 3, issue,
              lambda b: pltpu.make_async_copy(feat_hbm.at[sidx[b]], rows[b],
                                              semg[b]).wait(),
              lambda b: pltpu.sync_copy(rows[b], agg.at[didx[b]], add=True),
              lambda b: None)

        toff = base + nfull * C
        if tail == 16:
            pltpu.sync_copy(src_hbm.at[pl.ds(toff, 16)], sidx16_v)
            pltpu.sync_copy(dst_hbm.at[pl.ds(toff, 16)], didx16_v)
            pltpu.async_copy(feat_hbm.at[sidx16_v], rows[0].at[pl.ds(0, 16)], semg[0]).wait()
            pltpu.sync_copy(rows[0].at[pl.ds(0, 16)], agg.at[didx16_v], add=True)
        elif tail == 8:
            pltpu.sync_copy(src_hbm.at[pl.ds(toff, 8)], sidx8_v)
            pltpu.sync_copy(dst_hbm.at[pl.ds(toff, 8)], didx8_v)
            pltpu.async_copy(feat_hbm.at[sidx8_v], rows[0].at[pl.ds(0, 8)], semg[0]).wait()
            pltpu.sync_copy(rows[0].at[pl.ds(0, 8)], agg.at[didx8_v], add=True)

        plsc.subcore_barrier()

        _striped(s, 0, 8, n_dst,
                 lambda off, sz: pltpu.sync_copy(
                     agg.at[pl.ds(off, sz)],
                     out_hbm.at[pl.ds(c * n_dst + off, sz)]))

    return msg


_k0 = _make_k0()
_k1 = _make_k1()
_msg0 = _make_msg(N1, E0)
_msg1 = _make_msg(N2, E1)


def _scale_body(x_ref, d_ref, o_ref):
    d = jnp.maximum(d_ref[...], 1.0)
    o_ref[...] = x_ref[...] * (lax.rsqrt(d) * (1.0 / WORDS))


def _tc_scale(x, deg):
    grid = (N0 // 1000,)
    return pl.pallas_call(
        _scale_body,
        grid=grid,
        in_specs=[pl.BlockSpec((1000, D), lambda i: (i, 0)),
                  pl.BlockSpec((1000, 1), lambda i: (i, 0))],
        out_specs=pl.BlockSpec((1000, D), lambda i: (i, 0)),
        out_shape=jax.ShapeDtypeStruct((N0, D), jnp.float32),
    )(x, deg)


def _layer_body(scale_next, p0_ref, p1_ref, dd_ref, ds_ref, w_ref, b_ref, o_ref):
    dd = jnp.maximum(dd_ref[...], 1.0)
    agg = (p0_ref[...] + p1_ref[...]) * lax.rsqrt(dd)
    h = jnp.dot(agg, w_ref[...], preferred_element_type=jnp.float32) + b_ref[...]
    h = jnp.maximum(h, 0.0)
    if scale_next:
        ds = jnp.maximum(ds_ref[...], 1.0)
        h = h * lax.rsqrt(ds)
    o_ref[...] = h


def _tc_layer(p0, p1, deg_dst, deg_src_next, w, b, scale_next):
    n = p0.shape[0]
    blk = 1000
    grid = (n // blk,)
    return pl.pallas_call(
        functools.partial(_layer_body, scale_next),
        grid=grid,
        in_specs=[pl.BlockSpec((blk, D), lambda i: (i, 0)),
                  pl.BlockSpec((blk, D), lambda i: (i, 0)),
                  pl.BlockSpec((blk, 1), lambda i: (i, 0)),
                  pl.BlockSpec((blk, 1), lambda i: (i, 0)),
                  pl.BlockSpec((D, D), lambda i: (0, 0)),
                  pl.BlockSpec((1, D), lambda i: (0, 0))],
        out_specs=pl.BlockSpec((blk, D), lambda i: (i, 0)),
        out_shape=jax.ShapeDtypeStruct((n, D), jnp.float32),
    )(p0, p1, deg_dst, deg_src_next, w, b)


def _hist_to_deg(hp, rows, n):
    return (hp[:rows] + hp[rows:]).reshape(-1)[:n].reshape(n, 1)


def kernel(user_word, edge_index0, edge_index1, labels, word_table, W1, b1, W2, b2):
    uw_flat = user_word.reshape(-1).astype(jnp.int32)
    u = np.arange(UPT * WORDS, dtype=np.int32) // WORDS
    blocks = (u[None, :] + np.arange(NS, dtype=np.int32)[:, None] * UPT)
    uslot = np.tile(blocks.reshape(-1), NC)
    last = np.where(u >= 240, (NS - 1) * UPT + u - 240,
                    (NS - 1) * UPT + 80 + (u % 240)).astype(np.int32)
    uslot[(NW - 1) * UPT * WORDS:] = last

    src0 = edge_index0[0].astype(jnp.int32)
    dst0 = edge_index0[1].astype(jnp.int32)
    src1 = edge_index1[0].astype(jnp.int32)
    dst1 = edge_index1[1].astype(jnp.int32)

    z128 = np.zeros((128, D), np.float32)
    r80 = np.arange(R0S, dtype=np.int32)
    r40 = np.arange(R0D, dtype=np.int32)
    r8 = np.arange(R1D, dtype=np.int32)

    hp0s, hp0d, hp1s, hp1d = _k0(src0, dst0, src1, dst1, z128, r80, r40, r8)
    x_sum = _k1(word_table, uw_flat, uslot, z128)

    deg0s = _hist_to_deg(hp0s, R0S, N0)
    deg0d = _hist_to_deg(hp0d, R0D, N1)
    deg1s = _hist_to_deg(hp1s, R1S, N1)
    deg1d = _hist_to_deg(hp1d, R1D, N2)

    feat0 = _tc_scale(x_sum, deg0s)

    agg0 = _msg0(feat0, src0, dst0, z128)
    feat1 = _tc_layer(agg0[:N1], agg0[N1:], deg0d, deg1s, W1,
                      b1.reshape(1, D), True)

    agg1 = _msg1(feat1, src1, dst1, z128)
    out = _tc_layer(agg1[:N2], agg1[N2:], deg1d, deg1d, W2,
                    b2.reshape(1, D), False)

    return (out, labels)
